# baseline (speedup 1.0000x reference)
import functools

import jax
import jax.numpy as jnp
from jax import lax
from jax.experimental import pallas as pl
from jax.experimental.pallas import tpu as pltpu
from jax.experimental.pallas import tpu_sc as plsc

_NUM_EMB = 10
_EMB_DIM = 3
_ROWS = 50
_COLS = 16384
_NW = 32
_W = _COLS // _NW
_NVEC = _W // 16
_CHUNKS = [(0, 8), (8, 8), (16, 8), (24, 8), (32, 8), (40, 8), (48, 2)]


@functools.partial(
    pl.kernel,
    out_type=jax.ShapeDtypeStruct((_EMB_DIM, _ROWS, _COLS), jnp.float32),
    mesh=plsc.VectorSubcoreMesh(core_axis_name="c", subcore_axis_name="s"),
    compiler_params=pltpu.CompilerParams(needs_layout_passes=False),
    scratch_types=[
        pltpu.VMEM((_NUM_EMB, _EMB_DIM), jnp.float32),
        pltpu.VMEM((_ROWS, _W), jnp.int32),
        pltpu.VMEM((_EMB_DIM, _ROWS, _W), jnp.float32),
        [pltpu.SemaphoreType.DMA] * len(_CHUNKS),
        pltpu.SemaphoreType.DMA,
    ],
)
def _sc_gather(tbl_hbm, idx_hbm, out_hbm, tbl_v, idx_v, out_v, in_sems, sem_o):
    nc = 2
    wid = lax.axis_index("s") * nc + lax.axis_index("c")
    c0 = wid * _W

    in_cps = [
        pltpu.async_copy(
            idx_hbm.at[pl.ds(r0, nr), pl.ds(c0, _W)],
            idx_v.at[pl.ds(r0, nr)], in_sems[i])
        for i, (r0, nr) in enumerate(_CHUNKS)
    ]

    pltpu.sync_copy(tbl_hbm, tbl_v)
    lane = lax.iota(jnp.int32, 16)
    row = jnp.minimum(lane, _NUM_EMB - 1)
    tvals = [
        plsc.load_gather(tbl_v, [row, jnp.full((16,), d, jnp.int32)])
        for d in range(_EMB_DIM)
    ]

    out_cps = []
    for i, (r0, nr) in enumerate(_CHUNKS):
        in_cps[i].wait()

        def body(j, r0=r0):
            r = r0 + j // _NVEC
            k = (j % _NVEC) * 16
            c = idx_v[r, pl.ds(k, 16)] & 15
            for d in range(_EMB_DIM):
                out_v[d, r, pl.ds(k, 16)] = jnp.take_along_axis(
                    tvals[d], c, axis=0)

        plsc.parallel_loop(0, nr * _NVEC, unroll=8)(body)
        for d in range(_EMB_DIM):
            out_cps.append(pltpu.async_copy(
                out_v.at[d, pl.ds(r0, nr)],
                out_hbm.at[d, pl.ds(r0, nr), pl.ds(c0, _W)], sem_o))

    for cp in out_cps:
        cp.wait()


def kernel(arg, table):
    out = _sc_gather(table, arg.T)
    return out.transpose(2, 1, 0)


# speedup vs baseline: 1.0015x; 1.0015x over previous
import functools

import jax
import jax.numpy as jnp
from jax import lax
from jax.experimental import pallas as pl
from jax.experimental.pallas import tpu as pltpu
from jax.experimental.pallas import tpu_sc as plsc

_NUM_EMB = 10
_EMB_DIM = 3
_ROWS = 50
_COLS = 16384
_NW = 32
_W = _COLS // _NW
_NVEC = _W // 16
_CHUNKS = [(0, 8), (8, 8), (16, 8), (24, 8), (32, 8), (40, 8), (48, 2)]


@functools.partial(
    pl.kernel,
    out_type=jax.ShapeDtypeStruct((_EMB_DIM, _ROWS, _COLS), jnp.float32),
    mesh=plsc.VectorSubcoreMesh(core_axis_name="c", subcore_axis_name="s"),
    compiler_params=pltpu.CompilerParams(needs_layout_passes=False),
    scratch_types=[
        pltpu.VMEM((_NUM_EMB, _EMB_DIM), jnp.float32),
        pltpu.VMEM((_ROWS, _W), jnp.int32),
        pltpu.VMEM((_EMB_DIM, _ROWS, _W), jnp.float32),
        [pltpu.SemaphoreType.DMA] * len(_CHUNKS),
        pltpu.SemaphoreType.DMA,
    ],
)
def _sc_gather(tbl_hbm, idx_hbm, out_hbm, tbl_v, idx_v, out_v, in_sems, sem_o):
    nc = 2
    wid = lax.axis_index("s") * nc + lax.axis_index("c")
    c0 = wid * _W

    in_cps = [
        pltpu.async_copy(
            idx_hbm.at[pl.ds(r0, nr), pl.ds(c0, _W)],
            idx_v.at[pl.ds(r0, nr)], in_sems[i])
        for i, (r0, nr) in enumerate(_CHUNKS)
    ]

    with jax.named_scope("tbl_setup"):
        pltpu.sync_copy(tbl_hbm, tbl_v)
        lane = lax.iota(jnp.int32, 16)
        row = jnp.minimum(lane, _NUM_EMB - 1)
        tvals = [
            plsc.load_gather(tbl_v, [row, jnp.full((16,), d, jnp.int32)])
            for d in range(_EMB_DIM)
        ]

    out_cps = []
    for i, (r0, nr) in enumerate(_CHUNKS):
        with jax.named_scope(f"wait{i}"):
            in_cps[i].wait()

        def body(j, r0=r0):
            r = r0 + j // _NVEC
            k = (j % _NVEC) * 16
            c = idx_v[r, pl.ds(k, 16)] & 15
            for d in range(_EMB_DIM):
                out_v[d, r, pl.ds(k, 16)] = jnp.take_along_axis(
                    tvals[d], c, axis=0)

        with jax.named_scope(f"comp{i}"):
            plsc.parallel_loop(0, nr * _NVEC, unroll=8)(body)
        with jax.named_scope(f"oissue{i}"):
            for d in range(_EMB_DIM):
                out_cps.append(pltpu.async_copy(
                    out_v.at[d, pl.ds(r0, nr)],
                    out_hbm.at[d, pl.ds(r0, nr), pl.ds(c0, _W)], sem_o))

    with jax.named_scope("drain"):
        for cp in out_cps:
            cp.wait()


def kernel(arg, table):
    out = _sc_gather(table, arg.T)
    return out.transpose(2, 1, 0)


# speedup vs baseline: 1.0030x; 1.0015x over previous
import functools

import jax
import jax.numpy as jnp
from jax import lax
from jax.experimental import pallas as pl
from jax.experimental.pallas import tpu as pltpu
from jax.experimental.pallas import tpu_sc as plsc

_NUM_EMB = 10
_EMB_DIM = 3
_ROWS = 50
_COLS = 16384
_NW = 32
_W = _COLS // _NW
_NVEC = _W // 16
_TBL = 48
_CHUNKS = [(0, 8), (8, 8), (16, 8), (24, 8), (32, 8), (40, 8), (48, 2)]


@functools.partial(
    pl.kernel,
    out_type=jax.ShapeDtypeStruct((_EMB_DIM, _ROWS, _COLS), jnp.float32),
    mesh=plsc.VectorSubcoreMesh(core_axis_name="c", subcore_axis_name="s"),
    compiler_params=pltpu.CompilerParams(needs_layout_passes=False),
    scratch_types=[
        pltpu.VMEM((_TBL,), jnp.float32),
        pltpu.VMEM((_ROWS, _W), jnp.int32),
        pltpu.VMEM((_EMB_DIM, _ROWS, _W), jnp.float32),
        [pltpu.SemaphoreType.DMA] * len(_CHUNKS),
        pltpu.SemaphoreType.DMA,
    ],
)
def _sc_gather(tbl_hbm, idx_hbm, out_hbm, tbl_v, idx_v, out_v, in_sems, sem_o):
    nc = 2
    wid = lax.axis_index("s") * nc + lax.axis_index("c")
    c0 = wid * _W

    in_cps = [
        pltpu.async_copy(
            idx_hbm.at[pl.ds(r0, nr), pl.ds(c0, _W)],
            idx_v.at[pl.ds(r0, nr)], in_sems[i])
        for i, (r0, nr) in enumerate(_CHUNKS)
    ]

    pltpu.sync_copy(tbl_hbm, tbl_v)
    lane3 = lax.iota(jnp.int32, 16) * 3
    tvals = [plsc.load_gather(tbl_v, [lane3 + d]) for d in range(_EMB_DIM)]

    out_cps = []
    for i, (r0, nr) in enumerate(_CHUNKS):
        in_cps[i].wait()

        def body(j, r0=r0):
            r = r0 + j // _NVEC
            k = (j % _NVEC) * 16
            c = idx_v[r, pl.ds(k, 16)] & 15
            for d in range(_EMB_DIM):
                out_v[d, r, pl.ds(k, 16)] = jnp.take_along_axis(
                    tvals[d], c, axis=0)

        plsc.parallel_loop(0, nr * _NVEC, unroll=8)(body)
        for d in range(_EMB_DIM):
            out_cps.append(pltpu.async_copy(
                out_v.at[d, pl.ds(r0, nr)],
                out_hbm.at[d, pl.ds(r0, nr), pl.ds(c0, _W)], sem_o))

    for cp in out_cps:
        cp.wait()


def kernel(arg, table):
    tbl = jnp.pad(table.reshape(-1), (0, _TBL - _NUM_EMB * _EMB_DIM))
    out = _sc_gather(tbl, arg.T)
    return out.transpose(2, 1, 0)


# speedup vs baseline: 1.0276x; 1.0246x over previous
import functools

import jax
import jax.numpy as jnp
from jax import lax
from jax.experimental import pallas as pl
from jax.experimental.pallas import tpu as pltpu
from jax.experimental.pallas import tpu_sc as plsc

_NUM_EMB = 10
_EMB_DIM = 3
_ROWS = 50
_COLS = 16384
_NW = 32
_W = _COLS // _NW
_NVEC = _W // 16
_CHUNKS = [(0, 8), (8, 8), (16, 8), (24, 8), (32, 8), (40, 8), (48, 2)]


@functools.partial(
    pl.kernel,
    out_type=jax.ShapeDtypeStruct((_EMB_DIM, _ROWS, _COLS), jnp.float32),
    mesh=plsc.VectorSubcoreMesh(core_axis_name="c", subcore_axis_name="s"),
    compiler_params=pltpu.CompilerParams(needs_layout_passes=False),
    scratch_types=[
        pltpu.VMEM((_EMB_DIM, _NUM_EMB), jnp.float32),
        pltpu.VMEM((_ROWS, _W), jnp.int32),
        pltpu.VMEM((_EMB_DIM, _ROWS, _W), jnp.float32),
        [pltpu.SemaphoreType.DMA] * len(_CHUNKS),
        pltpu.SemaphoreType.DMA,
    ],
)
def _sc_gather(tbl_hbm, idx_hbm, out_hbm, tbl_v, idx_v, out_v, in_sems, sem_o):
    nc = 2
    wid = lax.axis_index("s") * nc + lax.axis_index("c")
    c0 = wid * _W

    in_cps = [
        pltpu.async_copy(
            idx_hbm.at[pl.ds(r0, nr), pl.ds(c0, _W)],
            idx_v.at[pl.ds(r0, nr)], in_sems[i])
        for i, (r0, nr) in enumerate(_CHUNKS)
    ]

    pltpu.sync_copy(tbl_hbm, tbl_v)
    lane = lax.iota(jnp.int32, 16)
    row = jnp.minimum(lane, _NUM_EMB - 1)
    tvals = [
        plsc.load_gather(tbl_v, [jnp.full((16,), d, jnp.int32), row])
        for d in range(_EMB_DIM)
    ]

    out_cps = []
    for i, (r0, nr) in enumerate(_CHUNKS):
        in_cps[i].wait()

        def body(j, r0=r0):
            r = r0 + j // _NVEC
            k = (j % _NVEC) * 16
            c = idx_v[r, pl.ds(k, 16)] & 15
            for d in range(_EMB_DIM):
                out_v[d, r, pl.ds(k, 16)] = jnp.take_along_axis(
                    tvals[d], c, axis=0)

        plsc.parallel_loop(0, nr * _NVEC, unroll=4)(body)
        for d in range(_EMB_DIM):
            out_cps.append(pltpu.async_copy(
                out_v.at[d, pl.ds(r0, nr)],
                out_hbm.at[d, pl.ds(r0, nr), pl.ds(c0, _W)], sem_o))

    for cp in out_cps:
        cp.wait()


def kernel(arg, table):
    out = _sc_gather(table.T, arg.T)
    return out.transpose(2, 1, 0)


# speedup vs baseline: 1.0852x; 1.0561x over previous
import functools

import jax
import jax.numpy as jnp
from jax import lax
from jax.experimental import pallas as pl
from jax.experimental.pallas import tpu as pltpu
from jax.experimental.pallas import tpu_sc as plsc

_NUM_EMB = 10
_EMB_DIM = 3
_ROWS = 50
_COLS = 16384
_NW = 32
_W = _COLS // _NW
_NVEC = _W // 16
_CHUNKS = [(0, 16), (16, 16), (32, 16), (48, 2)]


@functools.partial(
    pl.kernel,
    out_type=jax.ShapeDtypeStruct((_EMB_DIM, _ROWS, _COLS), jnp.float32),
    mesh=plsc.VectorSubcoreMesh(core_axis_name="c", subcore_axis_name="s"),
    compiler_params=pltpu.CompilerParams(needs_layout_passes=False),
    scratch_types=[
        pltpu.VMEM((_EMB_DIM, _NUM_EMB), jnp.float32),
        pltpu.VMEM((_ROWS, _W), jnp.int32),
        pltpu.VMEM((_EMB_DIM, _ROWS, _W), jnp.float32),
        [pltpu.SemaphoreType.DMA] * len(_CHUNKS),
        pltpu.SemaphoreType.DMA,
    ],
)
def _sc_gather(tbl_hbm, idx_hbm, out_hbm, tbl_v, idx_v, out_v, in_sems, sem_o):
    nc = 2
    wid = lax.axis_index("s") * nc + lax.axis_index("c")
    c0 = wid * _W

    in_cps = [
        pltpu.async_copy(
            idx_hbm.at[pl.ds(r0, nr), pl.ds(c0, _W)],
            idx_v.at[pl.ds(r0, nr)], in_sems[i])
        for i, (r0, nr) in enumerate(_CHUNKS)
    ]

    pltpu.sync_copy(tbl_hbm, tbl_v)
    lane = lax.iota(jnp.int32, 16)
    row = jnp.minimum(lane, _NUM_EMB - 1)
    tvals = [
        plsc.load_gather(tbl_v, [jnp.full((16,), d, jnp.int32), row])
        for d in range(_EMB_DIM)
    ]

    out_cps = []
    for i, (r0, nr) in enumerate(_CHUNKS):
        in_cps[i].wait()

        def body(j, r0=r0):
            r = r0 + j // _NVEC
            k = (j % _NVEC) * 16
            c = idx_v[r, pl.ds(k, 16)] & 15
            for d in range(_EMB_DIM):
                out_v[d, r, pl.ds(k, 16)] = jnp.take_along_axis(
                    tvals[d], c, axis=0)

        plsc.parallel_loop(0, nr * _NVEC, unroll=4)(body)
        for d in range(_EMB_DIM):
            out_cps.append(pltpu.async_copy(
                out_v.at[d, pl.ds(r0, nr)],
                out_hbm.at[d, pl.ds(r0, nr), pl.ds(c0, _W)], sem_o))

    for cp in out_cps:
        cp.wait()


def kernel(arg, table):
    out = _sc_gather(table.T, arg.T)
    return out.transpose(2, 1, 0)
